# baseline (speedup 1.0000x reference)
import functools

import jax
import jax.numpy as jnp
from jax import lax
from jax.experimental import pallas as pl
from jax.experimental.pallas import tpu as pltpu
from jax.experimental.pallas import tpu_sc as plsc

_N = 10000
_D = 128
_P = 200000

_L = 16
_BC = 128

_ROWS = 400


def _gcn_body(adj_ref, x_ref, w_ref, hp_ref, xw_ref):
    @pl.when(pl.program_id(0) == 0)
    def _():
        xw_ref[...] = jnp.dot(x_ref[...], w_ref[...],
                              preferred_element_type=jnp.float32)

    h = jnp.dot(adj_ref[...], xw_ref[...],
                preferred_element_type=jnp.float32)
    h = jnp.maximum(h, 0.0)
    ss = jnp.sum(h * h, axis=1, keepdims=True)
    hn = h * jax.lax.rsqrt(jnp.maximum(ss, 1e-12))
    hb = hn.astype(jnp.bfloat16)
    lo = jax.lax.bitcast_convert_type(hb[:, : _D // 2], jnp.int16)
    hi = jax.lax.bitcast_convert_type(hb[:, _D // 2:], jnp.int16)
    hp_ref[...] = (lo.astype(jnp.int32) & 0xFFFF) | (hi.astype(jnp.int32) << 16)


def _gcn(x, adj, W):
    return pl.pallas_call(
        _gcn_body,
        grid=(_N // _ROWS,),
        in_specs=[
            pl.BlockSpec((_ROWS, _N), lambda i: (i, 0)),
            pl.BlockSpec((_N, _D), lambda i: (0, 0)),
            pl.BlockSpec((_D, _D), lambda i: (0, 0)),
        ],
        out_specs=pl.BlockSpec((_ROWS, _D // 2), lambda i: (i, 0)),
        out_shape=jax.ShapeDtypeStruct((_N, _D // 2), jnp.int32),
        scratch_shapes=[pltpu.VMEM((_N, _D), jnp.float32)],
    )(adj, x, W)


def _sc_cosine(h, n1p, n2p, p_pad):
    info = plsc.get_sparse_core_info()
    nw = info.num_cores * info.num_subcores
    per_w = p_pad // nw
    n_chunks = per_w // _BC
    assert n_chunks % 2 == 0
    mesh = plsc.VectorSubcoreMesh(core_axis_name="c", subcore_axis_name="s")

    @functools.partial(
        pl.kernel,
        mesh=mesh,
        out_type=jax.ShapeDtypeStruct((p_pad,), jnp.float32),
        compiler_params=pltpu.CompilerParams(
            needs_layout_passes=False, use_tc_tiling_on_sc=False),
        scratch_types=[
            pltpu.VMEM((per_w,), jnp.int32),
            pltpu.VMEM((per_w,), jnp.int32),
            pltpu.VMEM((2, _BC, _D // 2), jnp.int32),
            pltpu.VMEM((2, _BC, _D // 2), jnp.int32),
            pltpu.VMEM((2, _L, _L), jnp.float32),
            pltpu.VMEM((per_w,), jnp.float32),
            pltpu.VMEM((_N // 80, _D // 2), jnp.int32),
            pltpu.VMEM_SHARED((_N, _D // 2), jnp.int32),
            pltpu.SemaphoreType.DMA,
            pltpu.SemaphoreType.DMA,
            pltpu.SemaphoreType.DMA,
            pltpu.SemaphoreType.DMA,
        ],
    )
    def k(h_hbm, i1_hbm, i2_hbm, out_hbm,
          i1_v, i2_v, r1_v, r2_v, m_v, o_v, stage_v, hs_sp,
          s1a, s1b, s2a, s2b):
        sid = lax.axis_index("s")
        wid = sid * info.num_cores + lax.axis_index("c")
        base = wid * per_w
        rows_pt = _N // info.num_subcores
        stage_rows = _N // 80
        tbase = sid * rows_pt
        for s in range(rows_pt // stage_rows):
            pltpu.sync_copy(
                h_hbm.at[pl.ds(tbase + s * stage_rows, stage_rows)], stage_v)
            pltpu.sync_copy(
                stage_v, hs_sp.at[pl.ds(tbase + s * stage_rows, stage_rows)])
        pltpu.sync_copy(i1_hbm.at[pl.ds(base, per_w)], i1_v)
        pltpu.sync_copy(i2_hbm.at[pl.ds(base, per_w)], i2_v)
        plsc.subcore_barrier()

        sems = ((s1a, s2a), (s1b, s2b))
        lanes = lax.iota(jnp.int32, _L)

        def gather_descr(c, slot):
            idx1 = i1_v.at[pl.ds(c * _BC, _BC)]
            idx2 = i2_v.at[pl.ds(c * _BC, _BC)]
            cp1 = pltpu.make_async_copy(
                hs_sp.at[idx1], r1_v.at[slot], sems[slot][0])
            cp2 = pltpu.make_async_copy(
                hs_sp.at[idx2], r2_v.at[slot], sems[slot][1])
            return cp1, cp2

        for slot in range(2):
            cp1, cp2 = gather_descr(jnp.int32(slot), slot)
            cp1.start()
            cp2.start()

        def compute_chunk(c, slot):
            cbase = c * _BC

            nt = _D // (2 * _L)

            def load_pair(p):
                return [(r1_v[slot, p, pl.ds(t * _L, _L)],
                         r2_v[slot, p, pl.ds(t * _L, _L)]) for t in range(nt)]

            def group_body(g, mb):
                gb = pl.multiple_of(cbase + g * _L, 8)
                gg = pl.multiple_of(g * _L, 8)
                cur = load_pair(gg)
                nx1 = load_pair(gg + 1)
                for j in range(_L):
                    nx2 = []
                    prods = []
                    for t in range(nt):
                        a, b = cur[t]
                        prods.append(plsc.bitcast(a, jnp.bfloat16)
                                     * plsc.bitcast(b, jnp.bfloat16))
                        if j + 2 < _L:
                            nx2.append((r1_v[slot, gg + j + 2, pl.ds(t * _L, _L)],
                                        r2_v[slot, gg + j + 2, pl.ds(t * _L, _L)]))
                    acc_bf = (prods[0] + prods[1]) + (prods[2] + prods[3])
                    lo, hi = plsc.unpack(
                        acc_bf, format=plsc.PackFormat.INTERLEAVED)
                    acc = lo + hi
                    plsc.store_scatter(
                        m_v.at[mb], [lanes, jnp.full((_L,), j, jnp.int32)], acc)
                    cur = nx1
                    nx1 = nx2
                rows = [m_v[mb, l, :] for l in range(_L)]
                while len(rows) > 1:
                    rows = [rows[i] + rows[i + 1]
                            for i in range(0, len(rows), 2)]
                o_v[pl.ds(gb, _L)] = rows[0]

            def group_pair(i, carry2):
                group_body(i * 2, 0)
                group_body(i * 2 + 1, 1)
                return carry2

            lax.fori_loop(0, _BC // (2 * _L), group_pair, 0)

        def pair_body(i, carry):
            c0 = i * 2
            for slot in range(2):
                c = c0 + slot
                cp1, cp2 = gather_descr(c, slot)
                cp1.wait()
                cp2.wait()
                compute_chunk(c, slot)
                nxt = c + 2

                @pl.when(nxt < n_chunks)
                def _():
                    np1, np2 = gather_descr(nxt, slot)
                    np1.start()
                    np2.start()
            return carry

        lax.fori_loop(0, n_chunks // 2, pair_body, 0)
        pltpu.sync_copy(o_v, out_hbm.at[pl.ds(base, per_w)])

    return k(h, n1p, n2p)


def kernel(x, adj, node1, node2, W):
    h = _gcn(x, adj, W)
    info = plsc.get_sparse_core_info()
    nw = info.num_cores * info.num_subcores
    quantum = nw * _BC * 2
    p_pad = ((_P + quantum - 1) // quantum) * quantum
    pad = p_pad - _P
    n1p = jnp.concatenate([node1, jnp.zeros((pad,), jnp.int32)])
    n2p = jnp.concatenate([node2, jnp.zeros((pad,), jnp.int32)])
    cos = _sc_cosine(h, n1p, n2p, p_pad)
    return cos[:_P].reshape(_P, 1)
